# Initial kernel scaffold; baseline (speedup 1.0000x reference)
import jax
import jax.numpy as jnp
from jax.experimental import pallas as pl


def kernel(x, edge_index, W1, b1, W2, b2):
    raise NotImplementedError("write your pallas kernel here")


# speedup vs baseline: 12.7722x; 12.7722x over previous
import dataclasses
import functools

import jax
import jax.numpy as jnp
from jax import lax
from jax.experimental import pallas as pl
from jax.experimental.pallas import tpu as pltpu
from jax.experimental.pallas import tpu_sc as plsc

NC = 2
NS = 16
NW = NC * NS
B = 128

N = 10000
E = 320000
K = 80
K2 = 2 * K
E_PAD = NW * K * B
N_PAD = 10112
RPT = N_PAD // NS


def _mesh():
    return plsc.VectorSubcoreMesh(core_axis_name="c", subcore_axis_name="s")


def _sc_params():
    cp = pltpu.CompilerParams()
    fields = pltpu.CompilerParams.__dataclass_fields__
    if "needs_layout_passes" in fields:
        cp = dataclasses.replace(cp, needs_layout_passes=False)
    if "use_tc_tiling_on_sc" in fields:
        cp = dataclasses.replace(cp, use_tc_tiling_on_sc=False)
    return cp


def _deg_pass(dst3d):
    @functools.partial(
        pl.kernel,
        out_type=jax.ShapeDtypeStruct((NW, N_PAD), jnp.float32),
        mesh=_mesh(),
        scratch_types=[
            pltpu.VMEM((K, B), jnp.int32),
            pltpu.VMEM((N_PAD,), jnp.float32),
        ],
        compiler_params=_sc_params(),
    )
    def k(dst_hbm, out_hbm, dst_v, hist):
        c = lax.axis_index("c")
        s = lax.axis_index("s")
        wid = c * NS + s
        pltpu.sync_copy(dst_hbm.at[wid], dst_v)

        def zero(i, carry):
            hist[pl.ds(i * 16, 16)] = jnp.zeros((16,), jnp.float32)
            return carry

        lax.fori_loop(0, N_PAD // 16, zero, 0)
        ones16 = jnp.ones((16,), jnp.float32)

        def body(j, carry):
            for l in range(B // 16):
                idx = dst_v[j, pl.ds(l * 16, 16)]
                plsc.addupdate_scatter(hist, [idx], ones16)
            return carry

        lax.fori_loop(0, K, body, 0)
        pltpu.sync_copy(hist, out_hbm.at[wid])

    return k(dst3d)


def _stream_loop(tbl_hbm, src_v, dst_v, r0, r1, acc, sem0, sem1, nsteps):
    pltpu.async_copy(tbl_hbm.at[src_v.at[0]], r0, sem0)
    plsc.subcore_barrier()

    def body(i, carry):
        j0 = 2 * i
        j1 = j0 + 1
        jn = jnp.minimum(j0 + 2, nsteps - 1)
        pltpu.make_async_copy(tbl_hbm.at[src_v.at[j0]], r0, sem0).wait()
        pltpu.async_copy(tbl_hbm.at[src_v.at[j1]], r1, sem1)
        pltpu.sync_copy(r0, acc.at[dst_v.at[j0]], add=True)
        pltpu.make_async_copy(tbl_hbm.at[src_v.at[j1]], r1, sem1).wait()
        pltpu.async_copy(tbl_hbm.at[src_v.at[jn]], r0, sem0)
        pltpu.sync_copy(r1, acc.at[dst_v.at[j1]], add=True)
        return carry

    lax.fori_loop(0, nsteps // 2, body, 0)
    pltpu.make_async_copy(tbl_hbm.at[src_v.at[nsteps - 1]], r0, sem0).wait()
    plsc.subcore_barrier()


def _seg_pass_feat(table2, srcf, dst2, zero_rows, d):
    @functools.partial(
        pl.kernel,
        out_type=jax.ShapeDtypeStruct((NC, N_PAD, d), jnp.float32),
        mesh=_mesh(),
        scratch_types=[
            pltpu.VMEM((K2, B), jnp.int32),
            pltpu.VMEM((K2, B), jnp.int32),
            pltpu.VMEM((B, d), jnp.float32),
            pltpu.VMEM((B, d), jnp.float32),
            pltpu.VMEM_SHARED((N_PAD, d), jnp.float32),
            pltpu.SemaphoreType.DMA,
            pltpu.SemaphoreType.DMA,
        ],
        compiler_params=_sc_params(),
    )
    def k(tbl_hbm, src_hbm, dst_hbm, zeros_hbm, out_hbm,
          src_v, dst_v, r0, r1, acc, sem0, sem1):
        c = lax.axis_index("c")
        s = lax.axis_index("s")
        pltpu.sync_copy(zeros_hbm, acc.at[pl.ds(s * RPT, RPT)])
        pltpu.sync_copy(src_hbm.at[c, s], src_v)
        pltpu.sync_copy(dst_hbm.at[s], dst_v)
        _stream_loop(tbl_hbm, src_v, dst_v, r0, r1, acc, sem0, sem1, K2)
        pltpu.sync_copy(acc.at[pl.ds(s * RPT, RPT)],
                        out_hbm.at[c, pl.ds(s * RPT, RPT)])

    return k(table2, srcf, dst2, zero_rows)


def _seg_pass_edge(table, src3d, dst3d, zero_rows, d):
    @functools.partial(
        pl.kernel,
        out_type=jax.ShapeDtypeStruct((NC, N_PAD, d), jnp.float32),
        mesh=_mesh(),
        scratch_types=[
            pltpu.VMEM((K, B), jnp.int32),
            pltpu.VMEM((K, B), jnp.int32),
            pltpu.VMEM((B, d), jnp.float32),
            pltpu.VMEM((B, d), jnp.float32),
            pltpu.VMEM_SHARED((N_PAD, d), jnp.float32),
            pltpu.SemaphoreType.DMA,
            pltpu.SemaphoreType.DMA,
        ],
        compiler_params=_sc_params(),
    )
    def k(tbl_hbm, src_hbm, dst_hbm, zeros_hbm, out_hbm,
          src_v, dst_v, r0, r1, acc, sem0, sem1):
        c = lax.axis_index("c")
        s = lax.axis_index("s")
        wid = c * NS + s
        pltpu.sync_copy(zeros_hbm, acc.at[pl.ds(s * RPT, RPT)])
        pltpu.sync_copy(src_hbm.at[wid], src_v)
        pltpu.sync_copy(dst_hbm.at[wid], dst_v)
        _stream_loop(tbl_hbm, src_v, dst_v, r0, r1, acc, sem0, sem1, K)
        pltpu.sync_copy(acc.at[pl.ds(s * RPT, RPT)],
                        out_hbm.at[c, pl.ds(s * RPT, RPT)])

    return k(table, src3d, dst3d, zero_rows)


def _dinv_block(deg_ref):
    return lax.rsqrt(jnp.sum(deg_ref[...], axis=1, keepdims=True) + 1.0)


_ROWS = 1000


def _k1(x, w1, degt):
    def body(x_ref, w_ref, deg_ref, o_ref):
        p = jax.lax.dot_general(x_ref[...], w_ref[...], (((1,), (0,)), ((), ())),
                                preferred_element_type=jnp.float32,
                                precision=jax.lax.Precision.HIGHEST)
        g = p * _dinv_block(deg_ref)
        h = w_ref.shape[1] // 2
        o_ref[0] = g[:, :h]
        o_ref[1] = g[:, h:]

    d_in, d_h = w1.shape
    return pl.pallas_call(
        body,
        grid=(N // _ROWS,),
        in_specs=[
            pl.BlockSpec((_ROWS, d_in), lambda i: (i, 0)),
            pl.BlockSpec((d_in, d_h), lambda i: (0, 0)),
            pl.BlockSpec((_ROWS, NW), lambda i: (i, 0)),
        ],
        out_specs=pl.BlockSpec((2, _ROWS, d_h // 2), lambda i: (0, i, 0)),
        out_shape=jax.ShapeDtypeStruct((2, N, d_h // 2), jnp.float32),
    )(x, w1, degt)


def _k2(s1, g1s, degt, b1, w2):
    def body(s_ref, g_ref, deg_ref, b_ref, w_ref, o_ref):
        dinv = _dinv_block(deg_ref)
        full = jnp.concatenate([s_ref[0] + g_ref[0], s_ref[1] + g_ref[1]],
                               axis=-1)
        h = dinv * full + b_ref[...]
        h = jnp.maximum(h, 0.0)
        p = jax.lax.dot_general(h, w_ref[...], (((1,), (0,)), ((), ())),
                                preferred_element_type=jnp.float32,
                                precision=jax.lax.Precision.HIGHEST)
        o_ref[...] = p * dinv

    d_h, d_o = w2.shape
    return pl.pallas_call(
        body,
        grid=(N // _ROWS,),
        in_specs=[
            pl.BlockSpec((2, _ROWS, d_h // 2), lambda i: (0, i, 0)),
            pl.BlockSpec((2, _ROWS, d_h // 2), lambda i: (0, i, 0)),
            pl.BlockSpec((_ROWS, NW), lambda i: (i, 0)),
            pl.BlockSpec((1, d_h), lambda i: (0, 0)),
            pl.BlockSpec((d_h, d_o), lambda i: (0, 0)),
        ],
        out_specs=pl.BlockSpec((_ROWS, d_o), lambda i: (i, 0)),
        out_shape=jax.ShapeDtypeStruct((N, d_o), jnp.float32),
    )(s1, g1s, degt, b1, w2)


def _k3(s2a, s2b, g2, degt, b2):
    def body(sa_ref, sb_ref, g_ref, deg_ref, b_ref, o_ref):
        dinv = _dinv_block(deg_ref)
        o_ref[...] = dinv * (sa_ref[...] + sb_ref[...] + g_ref[...]) + b_ref[...]

    d_o = s2a.shape[1]
    return pl.pallas_call(
        body,
        grid=(N // _ROWS,),
        in_specs=[
            pl.BlockSpec((_ROWS, d_o), lambda i: (i, 0)),
            pl.BlockSpec((_ROWS, d_o), lambda i: (i, 0)),
            pl.BlockSpec((_ROWS, d_o), lambda i: (i, 0)),
            pl.BlockSpec((_ROWS, NW), lambda i: (i, 0)),
            pl.BlockSpec((1, d_o), lambda i: (0, 0)),
        ],
        out_specs=pl.BlockSpec((_ROWS, d_o), lambda i: (i, 0)),
        out_shape=jax.ShapeDtypeStruct((N, d_o), jnp.float32),
    )(s2a, s2b, g2, degt, b2)


def kernel(x, edge_index, W1, b1, W2, b2):
    d_h = W1.shape[1]
    d_o = W2.shape[1]
    pad = E_PAD - E
    src = jnp.concatenate([edge_index[0], jnp.zeros((pad,), jnp.int32)])
    dst = jnp.concatenate([edge_index[1], jnp.full((pad,), N, jnp.int32)])
    src3d = src.reshape(NW, K, B)
    dst3d = dst.reshape(NW, K, B)
    src2 = src.reshape(NS, K2, B)
    srcf = jnp.stack([src2, src2 + N])
    dst2 = dst.reshape(NS, K2, B)

    zeros_h = jnp.zeros((RPT, d_h // 2), jnp.float32)
    zeros_o = jnp.zeros((RPT, d_o), jnp.float32)

    deg_parts = _deg_pass(dst3d)
    degt = deg_parts.T[:N, :]

    g1s = _k1(x, W1, degt)
    g1f = g1s.reshape(2 * N, d_h // 2)
    s1 = _seg_pass_feat(g1f, srcf, dst2, zeros_h, d_h // 2)
    g2 = _k2(s1[:, :N, :], g1s, degt, b1.reshape(1, d_h), W2)
    s2 = _seg_pass_edge(g2, src3d, dst3d, zeros_o, d_o)
    return _k3(s2[0, :N], s2[1, :N], g2, degt, b2.reshape(1, d_o))


# speedup vs baseline: 14.4977x; 1.1351x over previous
import dataclasses
import functools

import jax
import jax.numpy as jnp
from jax import lax
from jax.experimental import pallas as pl
from jax.experimental.pallas import tpu as pltpu
from jax.experimental.pallas import tpu_sc as plsc

NC = 2
NS = 16
NW = NC * NS
B = 128

N = 10000
E = 320000
K = 80
K2 = 2 * K
E_PAD = NW * K * B
N_PAD = 10112
RPT = N_PAD // NS


def _mesh():
    return plsc.VectorSubcoreMesh(core_axis_name="c", subcore_axis_name="s")


def _sc_params():
    cp = pltpu.CompilerParams()
    fields = pltpu.CompilerParams.__dataclass_fields__
    if "needs_layout_passes" in fields:
        cp = dataclasses.replace(cp, needs_layout_passes=False)
    if "use_tc_tiling_on_sc" in fields:
        cp = dataclasses.replace(cp, use_tc_tiling_on_sc=False)
    return cp


def _deg_pass(dst3d):
    @functools.partial(
        pl.kernel,
        out_type=jax.ShapeDtypeStruct((NW, N_PAD), jnp.float32),
        mesh=_mesh(),
        scratch_types=[
            pltpu.VMEM((K, B), jnp.int32),
            pltpu.VMEM((N_PAD,), jnp.float32),
        ],
        compiler_params=_sc_params(),
    )
    def k(dst_hbm, out_hbm, dst_v, hist):
        c = lax.axis_index("c")
        s = lax.axis_index("s")
        wid = c * NS + s
        pltpu.sync_copy(dst_hbm.at[wid], dst_v)

        def zero(i, carry):
            hist[pl.ds(i * 16, 16)] = jnp.zeros((16,), jnp.float32)
            return carry

        lax.fori_loop(0, N_PAD // 16, zero, 0)
        ones16 = jnp.ones((16,), jnp.float32)

        def body(j, carry):
            for l in range(B // 16):
                idx = dst_v[j, pl.ds(l * 16, 16)]
                plsc.addupdate_scatter(hist, [idx], ones16)
            return carry

        lax.fori_loop(0, K, body, 0)
        pltpu.sync_copy(hist, out_hbm.at[wid])

    return k(dst3d)


NBUF = 5


def _stream_loop(tbl_hbm, src_v, dst_v, bufs, acc, semg, sems, nsteps):
    ni = nsteps // NBUF
    for b in range(NBUF):
        pltpu.async_copy(tbl_hbm.at[src_v.at[b]], bufs[b], semg.at[b])
    plsc.subcore_barrier()

    def body(i, carry):
        for b in range(NBUF):
            j = i * NBUF + b
            pltpu.make_async_copy(tbl_hbm.at[src_v.at[j]], bufs[b],
                                  semg.at[b]).wait()
            pltpu.async_copy(bufs[b], acc.at[dst_v.at[j]], sems.at[b],
                             add=True)
        for b in range(NBUF):
            j = i * NBUF + b
            pltpu.make_async_copy(bufs[b], acc.at[dst_v.at[j]],
                                  sems.at[b]).wait()
            pltpu.async_copy(tbl_hbm.at[src_v.at[j + NBUF]], bufs[b],
                             semg.at[b])
        return carry

    lax.fori_loop(0, ni - 1, body, 0)
    for b in range(NBUF):
        j = (ni - 1) * NBUF + b
        pltpu.make_async_copy(tbl_hbm.at[src_v.at[j]], bufs[b],
                              semg.at[b]).wait()
        pltpu.async_copy(bufs[b], acc.at[dst_v.at[j]], sems.at[b], add=True)
    for b in range(NBUF):
        j = (ni - 1) * NBUF + b
        pltpu.make_async_copy(bufs[b], acc.at[dst_v.at[j]], sems.at[b]).wait()
    plsc.subcore_barrier()


def _seg_pass_feat(table2, srcf, dst2, zero_rows, d):
    @functools.partial(
        pl.kernel,
        out_type=jax.ShapeDtypeStruct((NC, N_PAD, d), jnp.float32),
        mesh=_mesh(),
        scratch_types=[
            pltpu.VMEM((K2, B), jnp.int32),
            pltpu.VMEM((K2, B), jnp.int32),
            pltpu.VMEM((NBUF, B, d), jnp.float32),
            pltpu.VMEM_SHARED((N_PAD, d), jnp.float32),
            pltpu.SemaphoreType.DMA((NBUF,)),
            pltpu.SemaphoreType.DMA((NBUF,)),
        ],
        compiler_params=_sc_params(),
    )
    def k(tbl_hbm, src_hbm, dst_hbm, zeros_hbm, out_hbm,
          src_v, dst_v, rows, acc, semg, sems):
        c = lax.axis_index("c")
        s = lax.axis_index("s")
        pltpu.sync_copy(zeros_hbm, acc.at[pl.ds(s * RPT, RPT)])
        pltpu.sync_copy(src_hbm.at[c, s], src_v)
        pltpu.sync_copy(dst_hbm.at[s], dst_v)
        bufs = [rows.at[b] for b in range(NBUF)]
        _stream_loop(tbl_hbm, src_v, dst_v, bufs, acc, semg, sems, K2)
        pltpu.sync_copy(acc.at[pl.ds(s * RPT, RPT)],
                        out_hbm.at[c, pl.ds(s * RPT, RPT)])

    return k(table2, srcf, dst2, zero_rows)


def _seg_pass_edge(table, src3d, dst3d, zero_rows, d):
    @functools.partial(
        pl.kernel,
        out_type=jax.ShapeDtypeStruct((NC, N_PAD, d), jnp.float32),
        mesh=_mesh(),
        scratch_types=[
            pltpu.VMEM((K, B), jnp.int32),
            pltpu.VMEM((K, B), jnp.int32),
            pltpu.VMEM((NBUF, B, d), jnp.float32),
            pltpu.VMEM_SHARED((N_PAD, d), jnp.float32),
            pltpu.SemaphoreType.DMA((NBUF,)),
            pltpu.SemaphoreType.DMA((NBUF,)),
        ],
        compiler_params=_sc_params(),
    )
    def k(tbl_hbm, src_hbm, dst_hbm, zeros_hbm, out_hbm,
          src_v, dst_v, rows, acc, semg, sems):
        c = lax.axis_index("c")
        s = lax.axis_index("s")
        wid = c * NS + s
        pltpu.sync_copy(zeros_hbm, acc.at[pl.ds(s * RPT, RPT)])
        pltpu.sync_copy(src_hbm.at[wid], src_v)
        pltpu.sync_copy(dst_hbm.at[wid], dst_v)
        bufs = [rows.at[b] for b in range(NBUF)]
        _stream_loop(tbl_hbm, src_v, dst_v, bufs, acc, semg, sems, K)
        pltpu.sync_copy(acc.at[pl.ds(s * RPT, RPT)],
                        out_hbm.at[c, pl.ds(s * RPT, RPT)])

    return k(table, src3d, dst3d, zero_rows)


def _dinv_block(deg_ref):
    return lax.rsqrt(jnp.sum(deg_ref[...], axis=1, keepdims=True) + 1.0)


_ROWS = 1000


def _k1(x, w1, degt):
    def body(x_ref, w_ref, deg_ref, o_ref):
        p = jax.lax.dot_general(x_ref[...], w_ref[...], (((1,), (0,)), ((), ())),
                                preferred_element_type=jnp.float32,
                                precision=jax.lax.Precision.HIGHEST)
        g = p * _dinv_block(deg_ref)
        h = w_ref.shape[1] // 2
        o_ref[0] = g[:, :h]
        o_ref[1] = g[:, h:]

    d_in, d_h = w1.shape
    return pl.pallas_call(
        body,
        grid=(N // _ROWS,),
        in_specs=[
            pl.BlockSpec((_ROWS, d_in), lambda i: (i, 0)),
            pl.BlockSpec((d_in, d_h), lambda i: (0, 0)),
            pl.BlockSpec((_ROWS, NW), lambda i: (i, 0)),
        ],
        out_specs=pl.BlockSpec((2, _ROWS, d_h // 2), lambda i: (0, i, 0)),
        out_shape=jax.ShapeDtypeStruct((2, N, d_h // 2), jnp.float32),
    )(x, w1, degt)


def _k2(s1, g1s, degt, b1, w2):
    def body(s_ref, g_ref, deg_ref, b_ref, w_ref, o_ref):
        dinv = _dinv_block(deg_ref)
        full = jnp.concatenate([s_ref[0] + g_ref[0], s_ref[1] + g_ref[1]],
                               axis=-1)
        h = dinv * full + b_ref[...]
        h = jnp.maximum(h, 0.0)
        p = jax.lax.dot_general(h, w_ref[...], (((1,), (0,)), ((), ())),
                                preferred_element_type=jnp.float32,
                                precision=jax.lax.Precision.HIGHEST)
        o_ref[...] = p * dinv

    d_h, d_o = w2.shape
    return pl.pallas_call(
        body,
        grid=(N // _ROWS,),
        in_specs=[
            pl.BlockSpec((2, _ROWS, d_h // 2), lambda i: (0, i, 0)),
            pl.BlockSpec((2, _ROWS, d_h // 2), lambda i: (0, i, 0)),
            pl.BlockSpec((_ROWS, NW), lambda i: (i, 0)),
            pl.BlockSpec((1, d_h), lambda i: (0, 0)),
            pl.BlockSpec((d_h, d_o), lambda i: (0, 0)),
        ],
        out_specs=pl.BlockSpec((_ROWS, d_o), lambda i: (i, 0)),
        out_shape=jax.ShapeDtypeStruct((N, d_o), jnp.float32),
    )(s1, g1s, degt, b1, w2)


def _k3(s2a, s2b, g2, degt, b2):
    def body(sa_ref, sb_ref, g_ref, deg_ref, b_ref, o_ref):
        dinv = _dinv_block(deg_ref)
        o_ref[...] = dinv * (sa_ref[...] + sb_ref[...] + g_ref[...]) + b_ref[...]

    d_o = s2a.shape[1]
    return pl.pallas_call(
        body,
        grid=(N // _ROWS,),
        in_specs=[
            pl.BlockSpec((_ROWS, d_o), lambda i: (i, 0)),
            pl.BlockSpec((_ROWS, d_o), lambda i: (i, 0)),
            pl.BlockSpec((_ROWS, d_o), lambda i: (i, 0)),
            pl.BlockSpec((_ROWS, NW), lambda i: (i, 0)),
            pl.BlockSpec((1, d_o), lambda i: (0, 0)),
        ],
        out_specs=pl.BlockSpec((_ROWS, d_o), lambda i: (i, 0)),
        out_shape=jax.ShapeDtypeStruct((N, d_o), jnp.float32),
    )(s2a, s2b, g2, degt, b2)


def kernel(x, edge_index, W1, b1, W2, b2):
    d_h = W1.shape[1]
    d_o = W2.shape[1]
    pad = E_PAD - E
    src = jnp.concatenate([edge_index[0], jnp.zeros((pad,), jnp.int32)])
    dst = jnp.concatenate([edge_index[1], jnp.full((pad,), N, jnp.int32)])
    src3d = src.reshape(NW, K, B)
    dst3d = dst.reshape(NW, K, B)
    src2 = src.reshape(NS, K2, B)
    srcf = jnp.stack([src2, src2 + N])
    dst2 = dst.reshape(NS, K2, B)

    zeros_h = jnp.zeros((RPT, d_h // 2), jnp.float32)
    zeros_o = jnp.zeros((RPT, d_o), jnp.float32)

    deg_parts = _deg_pass(dst3d)
    degt = deg_parts.T[:N, :]

    g1s = _k1(x, W1, degt)
    g1f = g1s.reshape(2 * N, d_h // 2)
    s1 = _seg_pass_feat(g1f, srcf, dst2, zeros_h, d_h // 2)
    g2 = _k2(s1[:, :N, :], g1s, degt, b1.reshape(1, d_h), W2)
    s2 = _seg_pass_edge(g2, src3d, dst3d, zeros_o, d_o)
    return _k3(s2[0, :N], s2[1, :N], g2, degt, b2.reshape(1, d_o))


# speedup vs baseline: 22.8683x; 1.5774x over previous
import dataclasses
import functools

import jax
import jax.numpy as jnp
from jax import lax
from jax.experimental import pallas as pl
from jax.experimental.pallas import tpu as pltpu
from jax.experimental.pallas import tpu_sc as plsc

NC = 2
NS = 16
NW = NC * NS
B = 128

N = 10000
E = 320000
K = 80
K2 = 2 * K
E_PAD = NW * K * B
N_PAD = 10112
RPT = N_PAD // NS


def _mesh():
    return plsc.VectorSubcoreMesh(core_axis_name="c", subcore_axis_name="s")


def _sc_params():
    cp = pltpu.CompilerParams()
    fields = pltpu.CompilerParams.__dataclass_fields__
    if "needs_layout_passes" in fields:
        cp = dataclasses.replace(cp, needs_layout_passes=False)
    if "use_tc_tiling_on_sc" in fields:
        cp = dataclasses.replace(cp, use_tc_tiling_on_sc=False)
    return cp


def _deg_pass(dst3d):
    @functools.partial(
        pl.kernel,
        out_type=jax.ShapeDtypeStruct((NW, N_PAD), jnp.float32),
        mesh=_mesh(),
        scratch_types=[
            pltpu.VMEM((K, B), jnp.int32),
            pltpu.VMEM((N_PAD,), jnp.float32),
        ],
        compiler_params=_sc_params(),
    )
    def k(dst_hbm, out_hbm, dst_v, hist):
        c = lax.axis_index("c")
        s = lax.axis_index("s")
        wid = c * NS + s
        pltpu.sync_copy(dst_hbm.at[wid], dst_v)

        def zero(i, carry):
            hist[pl.ds(i * 16, 16)] = jnp.zeros((16,), jnp.float32)
            return carry

        lax.fori_loop(0, N_PAD // 16, zero, 0)
        ones16 = jnp.ones((16,), jnp.float32)

        def body(j, carry):
            for l in range(B // 16):
                idx = dst_v[j, pl.ds(l * 16, 16)]
                plsc.addupdate_scatter(hist, [idx], ones16)
            return carry

        lax.fori_loop(0, K, body, 0)
        pltpu.sync_copy(hist, out_hbm.at[wid])

    return k(dst3d)


W = 4
SB = 16
NTR = N // NS


def _sb_pipeline(tbl, acc, idx, rows, semg, sems):
    for b in range(W):
        pltpu.async_copy(tbl.at[idx.at[b, 0]], rows.at[b], semg.at[b])
    rounds = SB // W
    for r in range(rounds):
        for b in range(W):
            t = r * W + b
            pltpu.make_async_copy(tbl.at[idx.at[t, 0]], rows.at[b],
                                  semg.at[b]).wait()
            pltpu.async_copy(rows.at[b], acc.at[idx.at[t, 1]], sems.at[b],
                             add=True)
        for b in range(W):
            t = r * W + b
            if t + W < SB:
                pltpu.make_async_copy(rows.at[b], acc.at[idx.at[t, 1]],
                                      sems.at[b]).wait()
                pltpu.async_copy(tbl.at[idx.at[t + W, 0]], rows.at[b],
                                 semg.at[b])
    for b in range(W):
        t = (rounds - 1) * W + b
        pltpu.make_async_copy(rows.at[b], acc.at[idx.at[t, 1]],
                              sems.at[b]).wait()


def _seg_pass(table, cmb, zero_rows, d, nsb, feat):
    @functools.partial(
        pl.kernel,
        out_type=jax.ShapeDtypeStruct((NC, N_PAD, d), jnp.float32),
        mesh=_mesh(),
        scratch_types=[
            pltpu.VMEM((2, SB, 2, B), jnp.int32),
            pltpu.VMEM((W, B, d), jnp.float32),
            pltpu.VMEM_SHARED((N, d), jnp.float32),
            pltpu.VMEM_SHARED((N_PAD, d), jnp.float32),
            pltpu.SemaphoreType.DMA,
            pltpu.SemaphoreType.DMA((W,)),
            pltpu.SemaphoreType.DMA((W,)),
        ],
        compiler_params=_sc_params(),
    )
    def k(tbl_hbm, cmb_hbm, zeros_hbm, out_hbm,
          idxb, rows, tbl, acc, semi, semg, sems):
        c = lax.axis_index("c")
        s = lax.axis_index("s")
        grp = s if feat else c * NS + s
        base = c * N if feat else 0
        pltpu.sync_copy(tbl_hbm.at[pl.ds(base + s * NTR, NTR)],
                        tbl.at[pl.ds(s * NTR, NTR)])
        pltpu.sync_copy(zeros_hbm, acc.at[pl.ds(s * RPT, RPT)])
        pltpu.sync_copy(cmb_hbm.at[grp, pl.ds(0, SB)], idxb.at[0])
        plsc.subcore_barrier()

        def body(q, carry):
            pq = jnp.bitwise_and(q, 1)

            @pl.when(q < nsb - 1)
            def _start():
                pltpu.async_copy(cmb_hbm.at[grp, pl.ds((q + 1) * SB, SB)],
                                 idxb.at[1 - pq], semi)

            _sb_pipeline(tbl, acc, idxb.at[pq], rows, semg, sems)

            @pl.when(q < nsb - 1)
            def _wait():
                pltpu.make_async_copy(cmb_hbm.at[grp, pl.ds((q + 1) * SB, SB)],
                                      idxb.at[1 - pq], semi).wait()

            return carry

        lax.fori_loop(0, nsb, body, 0)
        plsc.subcore_barrier()
        pltpu.sync_copy(acc.at[pl.ds(s * RPT, RPT)],
                        out_hbm.at[c, pl.ds(s * RPT, RPT)])

    return k(table, cmb, zero_rows)


def _dinv_block(deg_ref):
    return lax.rsqrt(jnp.sum(deg_ref[...], axis=1, keepdims=True) + 1.0)


_ROWS = 1000


def _k1(x, w1, degt):
    def body(x_ref, w_ref, deg_ref, o_ref):
        p = jax.lax.dot_general(x_ref[...], w_ref[...], (((1,), (0,)), ((), ())),
                                preferred_element_type=jnp.float32,
                                precision=jax.lax.Precision.HIGHEST)
        g = p * _dinv_block(deg_ref)
        h = w_ref.shape[1] // 2
        o_ref[0] = g[:, :h]
        o_ref[1] = g[:, h:]

    d_in, d_h = w1.shape
    return pl.pallas_call(
        body,
        grid=(N // _ROWS,),
        in_specs=[
            pl.BlockSpec((_ROWS, d_in), lambda i: (i, 0)),
            pl.BlockSpec((d_in, d_h), lambda i: (0, 0)),
            pl.BlockSpec((_ROWS, NW), lambda i: (i, 0)),
        ],
        out_specs=pl.BlockSpec((2, _ROWS, d_h // 2), lambda i: (0, i, 0)),
        out_shape=jax.ShapeDtypeStruct((2, N, d_h // 2), jnp.float32),
    )(x, w1, degt)


def _k2(s1, g1s, degt, b1, w2):
    def body(s_ref, g_ref, deg_ref, b_ref, w_ref, o_ref):
        dinv = _dinv_block(deg_ref)
        full = jnp.concatenate([s_ref[0] + g_ref[0], s_ref[1] + g_ref[1]],
                               axis=-1)
        h = dinv * full + b_ref[...]
        h = jnp.maximum(h, 0.0)
        p = jax.lax.dot_general(h, w_ref[...], (((1,), (0,)), ((), ())),
                                preferred_element_type=jnp.float32,
                                precision=jax.lax.Precision.HIGHEST)
        o_ref[...] = p * dinv

    d_h, d_o = w2.shape
    return pl.pallas_call(
        body,
        grid=(N // _ROWS,),
        in_specs=[
            pl.BlockSpec((2, _ROWS, d_h // 2), lambda i: (0, i, 0)),
            pl.BlockSpec((2, _ROWS, d_h // 2), lambda i: (0, i, 0)),
            pl.BlockSpec((_ROWS, NW), lambda i: (i, 0)),
            pl.BlockSpec((1, d_h), lambda i: (0, 0)),
            pl.BlockSpec((d_h, d_o), lambda i: (0, 0)),
        ],
        out_specs=pl.BlockSpec((_ROWS, d_o), lambda i: (i, 0)),
        out_shape=jax.ShapeDtypeStruct((N, d_o), jnp.float32),
    )(s1, g1s, degt, b1, w2)


def _k3(s2a, s2b, g2, degt, b2):
    def body(sa_ref, sb_ref, g_ref, deg_ref, b_ref, o_ref):
        dinv = _dinv_block(deg_ref)
        o_ref[...] = dinv * (sa_ref[...] + sb_ref[...] + g_ref[...]) + b_ref[...]

    d_o = s2a.shape[1]
    return pl.pallas_call(
        body,
        grid=(N // _ROWS,),
        in_specs=[
            pl.BlockSpec((_ROWS, d_o), lambda i: (i, 0)),
            pl.BlockSpec((_ROWS, d_o), lambda i: (i, 0)),
            pl.BlockSpec((_ROWS, d_o), lambda i: (i, 0)),
            pl.BlockSpec((_ROWS, NW), lambda i: (i, 0)),
            pl.BlockSpec((1, d_o), lambda i: (0, 0)),
        ],
        out_specs=pl.BlockSpec((_ROWS, d_o), lambda i: (i, 0)),
        out_shape=jax.ShapeDtypeStruct((N, d_o), jnp.float32),
    )(s2a, s2b, g2, degt, b2)


def kernel(x, edge_index, W1, b1, W2, b2):
    d_h = W1.shape[1]
    d_o = W2.shape[1]
    pad = E_PAD - E
    src = jnp.concatenate([edge_index[0], jnp.zeros((pad,), jnp.int32)])
    dst = jnp.concatenate([edge_index[1], jnp.full((pad,), N, jnp.int32)])
    dst3d = dst.reshape(NW, K, B)
    cmb = jnp.stack([src.reshape(NS, K2, B), dst.reshape(NS, K2, B)],
                    axis=2)
    cmb_edge = cmb.reshape(NW, K, 2, B)

    zeros_h = jnp.zeros((RPT, d_h // 2), jnp.float32)
    zeros_o = jnp.zeros((RPT, d_o), jnp.float32)

    deg_parts = _deg_pass(dst3d)
    degt = deg_parts.T[:N, :]

    g1s = _k1(x, W1, degt)
    g1f = g1s.reshape(2 * N, d_h // 2)
    s1 = _seg_pass(g1f, cmb, zeros_h, d_h // 2, K2 // SB, True)
    g2 = _k2(s1[:, :N, :], g1s, degt, b1.reshape(1, d_h), W2)
    s2 = _seg_pass(g2, cmb_edge, zeros_o, d_o, K // SB, False)
    return _k3(s2[0, :N], s2[1, :N], g2, degt, b2.reshape(1, d_o))


# speedup vs baseline: 23.9431x; 1.0470x over previous
import dataclasses
import functools

import jax
import jax.numpy as jnp
from jax import lax
from jax.experimental import pallas as pl
from jax.experimental.pallas import tpu as pltpu
from jax.experimental.pallas import tpu_sc as plsc

NC = 2
NS = 16
NW = NC * NS
B = 128

N = 10000
E = 320000
K = 80
K2 = 2 * K
E_PAD = NW * K * B
N_PAD = 10112
RPT = N_PAD // NS


def _mesh():
    return plsc.VectorSubcoreMesh(core_axis_name="c", subcore_axis_name="s")


def _sc_params():
    cp = pltpu.CompilerParams()
    fields = pltpu.CompilerParams.__dataclass_fields__
    if "needs_layout_passes" in fields:
        cp = dataclasses.replace(cp, needs_layout_passes=False)
    if "use_tc_tiling_on_sc" in fields:
        cp = dataclasses.replace(cp, use_tc_tiling_on_sc=False)
    return cp


def _deg_pass(dst3d):
    @functools.partial(
        pl.kernel,
        out_type=jax.ShapeDtypeStruct((NW, N), jnp.float32),
        mesh=_mesh(),
        scratch_types=[
            pltpu.VMEM((K, B), jnp.int32),
            pltpu.VMEM((N_PAD,), jnp.float32),
        ],
        compiler_params=_sc_params(),
    )
    def k(dst_hbm, out_hbm, dst_v, hist):
        c = lax.axis_index("c")
        s = lax.axis_index("s")
        wid = c * NS + s
        pltpu.sync_copy(dst_hbm.at[wid], dst_v)

        def zero(i, carry):
            hist[pl.ds(i * 16, 16)] = jnp.zeros((16,), jnp.float32)
            return carry

        lax.fori_loop(0, N_PAD // 16, zero, 0)
        ones16 = jnp.ones((16,), jnp.float32)

        def body(j, carry):
            for l in range(B // 16):
                idx = dst_v[j, pl.ds(l * 16, 16)]
                plsc.addupdate_scatter(hist, [idx], ones16)
            return carry

        lax.fori_loop(0, K, body, 0)
        pltpu.sync_copy(hist.at[pl.ds(0, N)], out_hbm.at[wid])

    return k(dst3d)


W = 4
SB = 16
NTR = N // NS


def _sb_pipeline(tbl, acc, idx, rows, semg, sems):
    for b in range(W):
        pltpu.async_copy(tbl.at[idx.at[b, 0]], rows.at[b], semg.at[b])
    rounds = SB // W
    for r in range(rounds):
        for b in range(W):
            t = r * W + b
            pltpu.make_async_copy(tbl.at[idx.at[t, 0]], rows.at[b],
                                  semg.at[b]).wait()
            pltpu.async_copy(rows.at[b], acc.at[idx.at[t, 1]], sems.at[b],
                             add=True)
        for b in range(W):
            t = r * W + b
            if t + W < SB:
                pltpu.make_async_copy(rows.at[b], acc.at[idx.at[t, 1]],
                                      sems.at[b]).wait()
                pltpu.async_copy(tbl.at[idx.at[t + W, 0]], rows.at[b],
                                 semg.at[b])
    for b in range(W):
        t = (rounds - 1) * W + b
        pltpu.make_async_copy(rows.at[b], acc.at[idx.at[t, 1]],
                              sems.at[b]).wait()


def _seg_pass(table, cmb, zero_rows, d, nsb, feat):
    @functools.partial(
        pl.kernel,
        out_type=jax.ShapeDtypeStruct((NC, N, d), jnp.float32),
        mesh=_mesh(),
        scratch_types=[
            pltpu.VMEM((2, SB, 2, B), jnp.int32),
            pltpu.VMEM((W, B, d), jnp.float32),
            pltpu.VMEM_SHARED((N, d), jnp.float32),
            pltpu.VMEM_SHARED((N_PAD, d), jnp.float32),
            pltpu.SemaphoreType.DMA,
            pltpu.SemaphoreType.DMA((W,)),
            pltpu.SemaphoreType.DMA((W,)),
        ],
        compiler_params=_sc_params(),
    )
    def k(tbl_hbm, cmb_hbm, zeros_hbm, out_hbm,
          idxb, rows, tbl, acc, semi, semg, sems):
        c = lax.axis_index("c")
        s = lax.axis_index("s")
        grp = s if feat else c * NS + s
        base = c * N if feat else 0
        pltpu.sync_copy(tbl_hbm.at[pl.ds(base + s * NTR, NTR)],
                        tbl.at[pl.ds(s * NTR, NTR)])
        pltpu.sync_copy(zeros_hbm, acc.at[pl.ds(s * RPT, RPT)])
        pltpu.sync_copy(cmb_hbm.at[grp, pl.ds(0, SB)], idxb.at[0])
        plsc.subcore_barrier()

        def body(q, carry):
            pq = jnp.bitwise_and(q, 1)

            @pl.when(q < nsb - 1)
            def _start():
                pltpu.async_copy(cmb_hbm.at[grp, pl.ds((q + 1) * SB, SB)],
                                 idxb.at[1 - pq], semi)

            _sb_pipeline(tbl, acc, idxb.at[pq], rows, semg, sems)

            @pl.when(q < nsb - 1)
            def _wait():
                pltpu.make_async_copy(cmb_hbm.at[grp, pl.ds((q + 1) * SB, SB)],
                                      idxb.at[1 - pq], semi).wait()

            return carry

        lax.fori_loop(0, nsb, body, 0)
        plsc.subcore_barrier()
        pltpu.sync_copy(acc.at[pl.ds(s * NTR, NTR)],
                        out_hbm.at[c, pl.ds(s * NTR, NTR)])

    return k(table, cmb, zero_rows)


def _dinv_block(deg_ref):
    return lax.rsqrt(jnp.sum(deg_ref[...], axis=1, keepdims=True) + 1.0)


_ROWS = 1000


def _k0(x, w1):
    def body(x_ref, w_ref, o_ref):
        o_ref[...] = jax.lax.dot_general(
            x_ref[...], w_ref[...], (((1,), (0,)), ((), ())),
            preferred_element_type=jnp.float32,
            precision=jax.lax.Precision.HIGHEST)

    d_in, d_h = w1.shape
    return pl.pallas_call(
        body,
        grid=(N // _ROWS,),
        in_specs=[
            pl.BlockSpec((_ROWS, d_in), lambda i: (i, 0)),
            pl.BlockSpec((d_in, d_h), lambda i: (0, 0)),
        ],
        out_specs=pl.BlockSpec((_ROWS, d_h), lambda i: (i, 0)),
        out_shape=jax.ShapeDtypeStruct((N, d_h), jnp.float32),
    )(x, w1)


def _k1(p1, degt):
    def body(p_ref, deg_ref, o_ref):
        g = p_ref[...] * _dinv_block(deg_ref)
        h = p_ref.shape[1] // 2
        o_ref[0] = g[:, :h]
        o_ref[1] = g[:, h:]

    d_h = p1.shape[1]
    return pl.pallas_call(
        body,
        grid=(N // _ROWS,),
        in_specs=[
            pl.BlockSpec((_ROWS, d_h), lambda i: (i, 0)),
            pl.BlockSpec((_ROWS, NW), lambda i: (i, 0)),
        ],
        out_specs=pl.BlockSpec((2, _ROWS, d_h // 2), lambda i: (0, i, 0)),
        out_shape=jax.ShapeDtypeStruct((2, N, d_h // 2), jnp.float32),
    )(p1, degt)


def _k2(s1, g1s, degt, b1, w2):
    def body(s_ref, g_ref, deg_ref, b_ref, w_ref, o_ref):
        dinv = _dinv_block(deg_ref)
        full = jnp.concatenate([s_ref[0] + g_ref[0], s_ref[1] + g_ref[1]],
                               axis=-1)
        h = dinv * full + b_ref[...]
        h = jnp.maximum(h, 0.0)
        p = jax.lax.dot_general(h, w_ref[...], (((1,), (0,)), ((), ())),
                                preferred_element_type=jnp.float32,
                                precision=jax.lax.Precision.HIGHEST)
        o_ref[...] = p * dinv

    d_h, d_o = w2.shape
    return pl.pallas_call(
        body,
        grid=(N // _ROWS,),
        in_specs=[
            pl.BlockSpec((2, _ROWS, d_h // 2), lambda i: (0, i, 0)),
            pl.BlockSpec((2, _ROWS, d_h // 2), lambda i: (0, i, 0)),
            pl.BlockSpec((_ROWS, NW), lambda i: (i, 0)),
            pl.BlockSpec((1, d_h), lambda i: (0, 0)),
            pl.BlockSpec((d_h, d_o), lambda i: (0, 0)),
        ],
        out_specs=pl.BlockSpec((_ROWS, d_o), lambda i: (i, 0)),
        out_shape=jax.ShapeDtypeStruct((N, d_o), jnp.float32),
    )(s1, g1s, degt, b1, w2)


def _k3(s2, g2, degt, b2):
    def body(s_ref, g_ref, deg_ref, b_ref, o_ref):
        dinv = _dinv_block(deg_ref)
        o_ref[...] = dinv * (s_ref[0] + s_ref[1] + g_ref[...]) + b_ref[...]

    d_o = g2.shape[1]
    return pl.pallas_call(
        body,
        grid=(N // _ROWS,),
        in_specs=[
            pl.BlockSpec((2, _ROWS, d_o), lambda i: (0, i, 0)),
            pl.BlockSpec((_ROWS, d_o), lambda i: (i, 0)),
            pl.BlockSpec((_ROWS, NW), lambda i: (i, 0)),
            pl.BlockSpec((1, d_o), lambda i: (0, 0)),
        ],
        out_specs=pl.BlockSpec((_ROWS, d_o), lambda i: (i, 0)),
        out_shape=jax.ShapeDtypeStruct((N, d_o), jnp.float32),
    )(s2, g2, degt, b2)


def kernel(x, edge_index, W1, b1, W2, b2):
    d_h = W1.shape[1]
    d_o = W2.shape[1]
    pad = E_PAD - E
    src = jnp.concatenate([edge_index[0], jnp.zeros((pad,), jnp.int32)])
    dst = jnp.concatenate([edge_index[1], jnp.full((pad,), N, jnp.int32)])
    dst3d = dst.reshape(NW, K, B)
    cmb = jnp.stack([src.reshape(NS, K2, B), dst.reshape(NS, K2, B)],
                    axis=2)
    cmb_edge = cmb.reshape(NW, K, 2, B)

    zeros_h = jnp.zeros((RPT, d_h // 2), jnp.float32)
    zeros_o = jnp.zeros((RPT, d_o), jnp.float32)

    deg_parts = _deg_pass(dst3d)
    degt = deg_parts.T

    p1 = _k0(x, W1)
    g1s = _k1(p1, degt)
    g1f = g1s.reshape(2 * N, d_h // 2)
    s1 = _seg_pass(g1f, cmb, zeros_h, d_h // 2, K2 // SB, True)
    g2 = _k2(s1, g1s, degt, b1.reshape(1, d_h), W2)
    s2 = _seg_pass(g2, cmb_edge, zeros_o, d_o, K // SB, False)
    return _k3(s2, g2, degt, b2.reshape(1, d_o))


# speedup vs baseline: 25.1426x; 1.0501x over previous
import dataclasses
import functools

import jax
import jax.numpy as jnp
from jax import lax
from jax.experimental import pallas as pl
from jax.experimental.pallas import tpu as pltpu
from jax.experimental.pallas import tpu_sc as plsc

NC = 2
NS = 16
NW = NC * NS
B = 128

N = 10000
E = 320000
K = 80
K2 = 2 * K
E_PAD = NW * K * B
N_PAD = 10112
RPT = N_PAD // NS


def _mesh():
    return plsc.VectorSubcoreMesh(core_axis_name="c", subcore_axis_name="s")


def _sc_params():
    cp = pltpu.CompilerParams()
    fields = pltpu.CompilerParams.__dataclass_fields__
    if "needs_layout_passes" in fields:
        cp = dataclasses.replace(cp, needs_layout_passes=False)
    if "use_tc_tiling_on_sc" in fields:
        cp = dataclasses.replace(cp, use_tc_tiling_on_sc=False)
    return cp


def _deg_pass(dst3d):
    @functools.partial(
        pl.kernel,
        out_type=jax.ShapeDtypeStruct((NW, N), jnp.float32),
        mesh=_mesh(),
        scratch_types=[
            pltpu.VMEM((K, B), jnp.int32),
            pltpu.VMEM((N_PAD,), jnp.float32),
        ],
        compiler_params=_sc_params(),
    )
    def k(dst_hbm, out_hbm, dst_v, hist):
        c = lax.axis_index("c")
        s = lax.axis_index("s")
        wid = c * NS + s
        pltpu.sync_copy(dst_hbm.at[wid], dst_v)

        def zero(i, carry):
            hist[pl.ds(i * 16, 16)] = jnp.zeros((16,), jnp.float32)
            return carry

        lax.fori_loop(0, N_PAD // 16, zero, 0)
        ones16 = jnp.ones((16,), jnp.float32)

        def body(j, carry):
            for l in range(B // 16):
                idx = dst_v[j, pl.ds(l * 16, 16)]
                plsc.addupdate_scatter(hist, [idx], ones16)
            return carry

        lax.fori_loop(0, K, body, 0)
        pltpu.sync_copy(hist.at[pl.ds(0, N)], out_hbm.at[wid])

    return k(dst3d)


W = 4
SB = 16
NTR = N // NS


def _sb_pipeline(tbl, acc, idx, rows, semg, sems):
    for b in range(W):
        pltpu.async_copy(tbl.at[idx.at[b, 0]], rows.at[b], semg.at[b])
    rounds = SB // W
    for r in range(rounds):
        for b in range(W):
            t = r * W + b
            pltpu.make_async_copy(tbl.at[idx.at[t, 0]], rows.at[b],
                                  semg.at[b]).wait()
            pltpu.async_copy(rows.at[b], acc.at[idx.at[t, 1]], sems.at[b],
                             add=True)
        for b in range(W):
            t = r * W + b
            if t + W < SB:
                pltpu.make_async_copy(rows.at[b], acc.at[idx.at[t, 1]],
                                      sems.at[b]).wait()
                pltpu.async_copy(tbl.at[idx.at[t + W, 0]], rows.at[b],
                                 semg.at[b])
    for b in range(W):
        t = (rounds - 1) * W + b
        pltpu.make_async_copy(rows.at[b], acc.at[idx.at[t, 1]],
                              sems.at[b]).wait()


def _seg_pass(table, cmb, zero_rows, d, nsb, feat):
    @functools.partial(
        pl.kernel,
        out_type=jax.ShapeDtypeStruct((NC, N, d), jnp.float32),
        mesh=_mesh(),
        scratch_types=[
            pltpu.VMEM((2, SB, 2, B), jnp.int32),
            pltpu.VMEM((W, B, d), jnp.float32),
            pltpu.VMEM_SHARED((N, d), jnp.float32),
            pltpu.VMEM_SHARED((N_PAD, d), jnp.float32),
            pltpu.SemaphoreType.DMA,
            pltpu.SemaphoreType.DMA((W,)),
            pltpu.SemaphoreType.DMA((W,)),
        ],
        compiler_params=_sc_params(),
    )
    def k(tbl_hbm, cmb_hbm, zeros_hbm, out_hbm,
          idxb, rows, tbl, acc, semi, semg, sems):
        c = lax.axis_index("c")
        s = lax.axis_index("s")
        grp = s if feat else c * NS + s
        if feat:
            stage_src = tbl_hbm.at[c, pl.ds(s * NTR, NTR)]
        else:
            stage_src = tbl_hbm.at[pl.ds(s * NTR, NTR)]
        pltpu.sync_copy(stage_src, tbl.at[pl.ds(s * NTR, NTR)])
        pltpu.sync_copy(zeros_hbm, acc.at[pl.ds(s * RPT, RPT)])
        pltpu.sync_copy(cmb_hbm.at[grp, pl.ds(0, SB)], idxb.at[0])
        plsc.subcore_barrier()

        def body(q, carry):
            pq = jnp.bitwise_and(q, 1)

            @pl.when(q < nsb - 1)
            def _start():
                pltpu.async_copy(cmb_hbm.at[grp, pl.ds((q + 1) * SB, SB)],
                                 idxb.at[1 - pq], semi)

            _sb_pipeline(tbl, acc, idxb.at[pq], rows, semg, sems)

            @pl.when(q < nsb - 1)
            def _wait():
                pltpu.make_async_copy(cmb_hbm.at[grp, pl.ds((q + 1) * SB, SB)],
                                      idxb.at[1 - pq], semi).wait()

            return carry

        lax.fori_loop(0, nsb, body, 0)
        plsc.subcore_barrier()
        pltpu.sync_copy(acc.at[pl.ds(s * NTR, NTR)],
                        out_hbm.at[c, pl.ds(s * NTR, NTR)])

    return k(table, cmb, zero_rows)


_ROWS = 2000


def _k0(x, w1):
    def body(x_ref, w_ref, o_ref):
        o_ref[...] = jax.lax.dot_general(
            x_ref[...], w_ref[...], (((1,), (0,)), ((), ())),
            preferred_element_type=jnp.float32,
            precision=jax.lax.Precision.HIGHEST)

    d_in, d_h = w1.shape
    return pl.pallas_call(
        body,
        grid=(N // _ROWS,),
        in_specs=[
            pl.BlockSpec((_ROWS, d_in), lambda i: (i, 0)),
            pl.BlockSpec((d_in, d_h), lambda i: (0, 0)),
        ],
        out_specs=pl.BlockSpec((_ROWS, d_h), lambda i: (i, 0)),
        out_shape=jax.ShapeDtypeStruct((N, d_h), jnp.float32),
    )(x, w1)


def _k1(p1, deg_parts):
    def body(p_ref, deg_ref, o_ref, dinv_ref):
        d = jnp.sum(deg_ref[...], axis=1, keepdims=True) + 1.0
        dinv = lax.rsqrt(d)
        dinv_ref[...] = dinv
        g = p_ref[...] * dinv
        h = p_ref.shape[1] // 2
        o_ref[0] = g[:, :h]
        o_ref[1] = g[:, h:]

    d_h = p1.shape[1]
    return pl.pallas_call(
        body,
        grid=(N // _ROWS,),
        in_specs=[
            pl.BlockSpec((_ROWS, d_h), lambda i: (i, 0)),
            pl.BlockSpec((_ROWS, NW), lambda i: (i, 0)),
        ],
        out_specs=[
            pl.BlockSpec((2, _ROWS, d_h // 2), lambda i: (0, i, 0)),
            pl.BlockSpec((_ROWS, 1), lambda i: (i, 0)),
        ],
        out_shape=[
            jax.ShapeDtypeStruct((2, N, d_h // 2), jnp.float32),
            jax.ShapeDtypeStruct((N, 1), jnp.float32),
        ],
    )(p1, deg_parts)


def _k2(s1, g1s, dinv, b1, w2):
    def body(s_ref, g_ref, dinv_ref, b_ref, w_ref, o_ref):
        dv = dinv_ref[...]
        full = jnp.concatenate([s_ref[0] + g_ref[0], s_ref[1] + g_ref[1]],
                               axis=-1)
        h = dv * full + b_ref[...]
        h = jnp.maximum(h, 0.0)
        p = jax.lax.dot_general(h, w_ref[...], (((1,), (0,)), ((), ())),
                                preferred_element_type=jnp.float32,
                                precision=jax.lax.Precision.HIGHEST)
        o_ref[...] = p * dv

    d_h, d_o = w2.shape
    return pl.pallas_call(
        body,
        grid=(N // _ROWS,),
        in_specs=[
            pl.BlockSpec((2, _ROWS, d_h // 2), lambda i: (0, i, 0)),
            pl.BlockSpec((2, _ROWS, d_h // 2), lambda i: (0, i, 0)),
            pl.BlockSpec((_ROWS, 1), lambda i: (i, 0)),
            pl.BlockSpec((1, d_h), lambda i: (0, 0)),
            pl.BlockSpec((d_h, d_o), lambda i: (0, 0)),
        ],
        out_specs=pl.BlockSpec((_ROWS, d_o), lambda i: (i, 0)),
        out_shape=jax.ShapeDtypeStruct((N, d_o), jnp.float32),
    )(s1, g1s, dinv, b1, w2)


def _k3(s2, g2, dinv, b2):
    def body(s_ref, g_ref, dinv_ref, b_ref, o_ref):
        o_ref[...] = (dinv_ref[...] * (s_ref[0] + s_ref[1] + g_ref[...])
                      + b_ref[...])

    d_o = g2.shape[1]
    return pl.pallas_call(
        body,
        grid=(N // _ROWS,),
        in_specs=[
            pl.BlockSpec((2, _ROWS, d_o), lambda i: (0, i, 0)),
            pl.BlockSpec((_ROWS, d_o), lambda i: (i, 0)),
            pl.BlockSpec((_ROWS, 1), lambda i: (i, 0)),
            pl.BlockSpec((1, d_o), lambda i: (0, 0)),
        ],
        out_specs=pl.BlockSpec((_ROWS, d_o), lambda i: (i, 0)),
        out_shape=jax.ShapeDtypeStruct((N, d_o), jnp.float32),
    )(s2, g2, dinv, b2)


def kernel(x, edge_index, W1, b1, W2, b2):
    d_h = W1.shape[1]
    d_o = W2.shape[1]
    pad = E_PAD - E
    src = jnp.concatenate([edge_index[0], jnp.zeros((pad,), jnp.int32)])
    dst = jnp.concatenate([edge_index[1], jnp.full((pad,), N, jnp.int32)])
    dst3d = dst.reshape(NW, K, B)
    cmb = jnp.stack([src.reshape(NS, K2, B), dst.reshape(NS, K2, B)],
                    axis=2)
    cmb_edge = cmb.reshape(NW, K, 2, B)

    zeros_h = jnp.zeros((RPT, d_h // 2), jnp.float32)
    zeros_o = jnp.zeros((RPT, d_o), jnp.float32)

    deg_parts = _deg_pass(dst3d)
    degt = deg_parts.T

    p1 = _k0(x, W1)
    g1s, dinv = _k1(p1, degt)
    s1 = _seg_pass(g1s, cmb, zeros_h, d_h // 2, K2 // SB, True)
    g2 = _k2(s1, g1s, dinv, b1.reshape(1, d_h), W2)
    s2 = _seg_pass(g2, cmb_edge, zeros_o, d_o, K // SB, False)
    return _k3(s2, g2, dinv, b2.reshape(1, d_o))
